# baseline (speedup 1.0000x reference)
import functools

import numpy as np
import jax
import jax.numpy as jnp
from jax import lax
from jax.experimental import pallas as pl
from jax.experimental.pallas import tpu as pltpu
from jax.sharding import Mesh, PartitionSpec as P

try:
    from jax import shard_map as _shard_map
except ImportError:
    from jax.experimental.shard_map import shard_map as _shard_map


def _rot(a, k):
    if k == 0:
        return a
    n = a.shape[1]
    k = k % n
    return jnp.concatenate([a[:, k:], a[:, :k]], axis=1)


def _stats_kernel(x_ref, sum_ref, sq_ref):
    @pl.when(pl.program_id(0) == 0)
    def _():
        sum_ref[...] = jnp.zeros_like(sum_ref)
        sq_ref[...] = jnp.zeros_like(sq_ref)

    x = x_ref[0]
    sum_ref[...] = sum_ref[...] + jnp.sum(x, axis=1, keepdims=True)
    sq_ref[...] = sq_ref[...] + jnp.sum(x * x, axis=1, keepdims=True)


def _glow_kernel(H, W, ns, G,
                 x_ref, nb_ref, sc_ref, pT_ref,
                 w1_ref, b1_ref, w2_ref, b2_ref, w3_ref, b3_ref,
                 y_ref, ld_ref):
    C = x_ref.shape[1]
    HW = x_ref.shape[2]
    co = C - ns
    n3 = 2 * co

    pix = lax.broadcasted_iota(jnp.int32, (1, HW), 1)
    py = pix // W
    px = pix - py * W
    taps = [(dy - 1, dx - 1) for dy in range(3) for dx in range(3)]
    valids = [((py + ky >= 0) & (py + ky < H) & (px + kx >= 0) & (px + kx < W))
              for (ky, kx) in taps]
    zero_b = jnp.zeros((), jnp.bfloat16)

    for g in range(G):
        z = (x_ref[g] + nb_ref[...]) * sc_ref[...]
        zc = jnp.dot(pT_ref[...], z.astype(jnp.bfloat16),
                     preferred_element_type=jnp.float32)
        y_ref[g, 0:ns, :] = zc[0:ns, :]
        z_b = zc[ns:C, :]

        za = zc[0:ns, :].astype(jnp.bfloat16)
        cols = [jnp.where(v, _rot(za, ky * W + kx), zero_b)
                for (ky, kx), v in zip(taps, valids)]
        col = jnp.concatenate(cols, axis=0)
        h1 = jnp.dot(w1_ref[...], col, preferred_element_type=jnp.float32)
        h1 = jnp.maximum(h1 + b1_ref[...], 0.0)

        h2 = jnp.dot(w2_ref[...], h1.astype(jnp.bfloat16),
                     preferred_element_type=jnp.float32)
        h2 = jnp.maximum(h2 + b2_ref[...], 0.0)

        part = jnp.dot(w3_ref[...], h2.astype(jnp.bfloat16),
                       preferred_element_type=jnp.float32)
        acc = jnp.broadcast_to(b3_ref[...], (n3, HW))
        for j, ((ky, kx), v) in enumerate(zip(taps, valids)):
            pj = _rot(part[j * n3:(j + 1) * n3, :], ky * W + kx)
            acc = acc + jnp.where(v, pj, 0.0)

        s = jax.nn.sigmoid(acc[0:co, :] + 2.0)
        t = acc[co:n3, :]
        y_ref[g, ns:C, :] = s * z_b + t
        ld_ref[g] = jnp.sum(jnp.log(jnp.abs(s)), keepdims=True)


def _forward(x3, matrix, w1, b1, w2, b2, w3, b3, n_shards, axis):
    Bl, C, HW = x3.shape
    H = W = int(round(HW ** 0.5))
    N = n_shards * Bl * HW
    ns = C // 2
    co = C - ns
    hid = w1.shape[-1]

    ch_sum, ch_sq = pl.pallas_call(
        _stats_kernel,
        grid=(Bl,),
        out_shape=(jax.ShapeDtypeStruct((C, 1), jnp.float32),
                   jax.ShapeDtypeStruct((C, 1), jnp.float32)),
        in_specs=[pl.BlockSpec((1, C, HW), lambda b: (b, 0, 0))],
        out_specs=(pl.BlockSpec((C, 1), lambda b: (0, 0)),
                   pl.BlockSpec((C, 1), lambda b: (0, 0))),
        compiler_params=pltpu.CompilerParams(
            dimension_semantics=("arbitrary",)),
    )(x3)
    if axis is not None:
        ch_sum = lax.psum(ch_sum, axis)
        ch_sq = lax.psum(ch_sq, axis)

    mean = ch_sum / N
    var = jnp.maximum((ch_sq - N * mean * mean) / (N - 1), 0.0)
    scale = 1.0 / (jnp.sqrt(var) + 1e-9)
    neg_bias = -mean

    logabsdet = jnp.sum(jnp.log(jnp.sum(jnp.abs(matrix), axis=0)))
    ld_const = HW * (jnp.sum(jnp.log(jnp.abs(scale))) + logabsdet)

    pT = matrix.T.astype(jnp.bfloat16)
    w1T = w1.reshape(9 * ns, hid).T.astype(jnp.bfloat16)
    w2T = w2.T.astype(jnp.bfloat16)
    w3r = jnp.concatenate([w3[..., 0::2], w3[..., 1::2]], -1)
    w3T = jnp.transpose(w3r, (0, 1, 3, 2)).reshape(9 * 2 * co, hid)
    w3T = w3T.astype(jnp.bfloat16)
    b3r = jnp.concatenate([b3[0::2], b3[1::2]]).reshape(2 * co, 1)

    def const_spec(shape):
        return pl.BlockSpec(shape, lambda b, _s=len(shape): (0,) * _s)

    G = 4
    while Bl % G:
        G //= 2
    y3, ld_cpl = pl.pallas_call(
        functools.partial(_glow_kernel, H, W, ns, G),
        grid=(Bl // G,),
        out_shape=(jax.ShapeDtypeStruct((Bl, C, HW), jnp.float32),
                   jax.ShapeDtypeStruct((Bl, 1, 1), jnp.float32)),
        in_specs=[
            pl.BlockSpec((G, C, HW), lambda b: (b, 0, 0)),
            const_spec((C, 1)),
            const_spec((C, 1)),
            const_spec((C, C)),
            const_spec((hid, 9 * ns)),
            const_spec((hid, 1)),
            const_spec((hid, hid)),
            const_spec((hid, 1)),
            const_spec((9 * 2 * co, hid)),
            const_spec((2 * co, 1)),
        ],
        out_specs=(pl.BlockSpec((G, C, HW), lambda b: (b, 0, 0)),
                   pl.BlockSpec((G, 1, 1), lambda b: (b, 0, 0))),
        compiler_params=pltpu.CompilerParams(
            dimension_semantics=("arbitrary",),
            vmem_limit_bytes=100 * 1024 * 1024),
    )(x3, neg_bias, scale, pT, w1T, b1.reshape(hid, 1), w2T,
      b2.reshape(hid, 1), w3T, b3r)

    log_det = ld_const * jnp.ones((Bl,), jnp.float32) + ld_cpl[:, 0, 0]
    return y3, log_det


def kernel(x, matrix, w1, b1, w2, b2, w3, b3):
    B, C, H, W = x.shape
    x3 = x.reshape(B, C, H * W)

    devs = jax.devices()
    if len(devs) >= 2 and B % 2 == 0:
        mesh = Mesh(np.asarray(devs[:2]), ("d",))
        specs = dict(
            mesh=mesh,
            in_specs=(P("d"), P(), P(), P(), P(), P(), P(), P()),
            out_specs=(P("d"), P("d")),
        )
        fn = functools.partial(_forward, n_shards=2, axis="d")
        try:
            fwd = _shard_map(fn, check_vma=False, **specs)
        except TypeError:
            fwd = _shard_map(fn, check_rep=False, **specs)
        y3, log_det = fwd(x3, matrix, w1, b1, w2, b2, w3, b3)
    else:
        y3, log_det = _forward(x3, matrix, w1, b1, w2, b2, w3, b3,
                               n_shards=1, axis=None)

    return y3.reshape(B, C, H, W), log_det


# speedup vs baseline: 1.7071x; 1.7071x over previous
import functools

import numpy as np
import jax
import jax.numpy as jnp
from jax import lax
from jax.experimental import pallas as pl
from jax.experimental.pallas import tpu as pltpu
from jax.sharding import Mesh, PartitionSpec as P

try:
    from jax import shard_map as _shard_map
except ImportError:
    from jax.experimental.shard_map import shard_map as _shard_map


def _rot(a, k):
    if k == 0:
        return a
    n = a.shape[1]
    k = k % n
    return jnp.concatenate([a[:, k:], a[:, :k]], axis=1)


def _stats_kernel(x_ref, sum_ref, sq_ref):
    @pl.when(pl.program_id(0) == 0)
    def _():
        sum_ref[...] = jnp.zeros_like(sum_ref)
        sq_ref[...] = jnp.zeros_like(sq_ref)

    x = x_ref[0]
    sum_ref[...] = sum_ref[...] + jnp.sum(x, axis=1, keepdims=True)
    sq_ref[...] = sq_ref[...] + jnp.sum(x * x, axis=1, keepdims=True)


def _glow_kernel(H, W, ns, G,
                 x_ref, nb_ref, sc_ref, pT_ref,
                 w1_ref, b1_ref, w2_ref, b2_ref, w3_ref, b3_ref,
                 y_ref, ld_ref):
    C = x_ref.shape[1]
    HW = x_ref.shape[2]
    co = C - ns
    n3 = 2 * co

    pix = lax.broadcasted_iota(jnp.int32, (1, HW), 1)
    py = pix // W
    px = pix - py * W
    vrow = {ky: (py + ky >= 0) & (py + ky < H) for ky in (-1, 0, 1)}
    vcol = {kx: (px + kx >= 0) & (px + kx < W) for kx in (-1, 0, 1)}
    taps = [(dy - 1, dx - 1) for dy in range(3) for dx in range(3)]
    valids = [vrow[ky] & vcol[kx] for (ky, kx) in taps]
    zero_b = jnp.zeros((), jnp.bfloat16)

    for g in range(G):
        z = (x_ref[g] + nb_ref[...]) * sc_ref[...]
        zc = jnp.dot(pT_ref[...], z.astype(jnp.bfloat16),
                     preferred_element_type=jnp.float32)
        y_ref[g, 0:ns, :] = zc[0:ns, :]
        z_b = zc[ns:C, :]

        za = zc[0:ns, :].astype(jnp.bfloat16)
        cols = [jnp.where(v, _rot(za, ky * W + kx), zero_b)
                for (ky, kx), v in zip(taps, valids)]
        col = jnp.concatenate(cols, axis=0)
        h1 = jnp.dot(w1_ref[...], col, preferred_element_type=jnp.float32)
        h1 = jnp.maximum(h1 + b1_ref[...], 0.0)

        h2 = jnp.dot(w2_ref[...], h1.astype(jnp.bfloat16),
                     preferred_element_type=jnp.float32)
        h2 = jnp.maximum(h2 + b2_ref[...], 0.0)

        h2b = h2.astype(jnp.bfloat16)
        colx = jnp.concatenate(
            [jnp.where(vcol[kx], _rot(h2b, kx), zero_b) if kx else h2b
             for kx in (-1, 0, 1)], axis=0)
        part = jnp.dot(w3_ref[...], colx,
                       preferred_element_type=jnp.float32)
        acc = jnp.broadcast_to(b3_ref[...], (n3, HW))
        for iy, ky in enumerate((-1, 0, 1)):
            pj = _rot(part[iy * n3:(iy + 1) * n3, :], ky * W)
            acc = acc + jnp.where(vrow[ky], pj, 0.0)

        s = jax.nn.sigmoid(acc[0:co, :] + 2.0)
        t = acc[co:n3, :]
        y_ref[g, ns:C, :] = s * z_b + t
        ld_ref[g] = jnp.sum(jnp.log(jnp.abs(s)), keepdims=True)


def _forward(x3, matrix, w1, b1, w2, b2, w3, b3, n_shards, axis):
    Bl, C, HW = x3.shape
    H = W = int(round(HW ** 0.5))
    N = n_shards * Bl * HW
    ns = C // 2
    co = C - ns
    hid = w1.shape[-1]

    ch_sum, ch_sq = pl.pallas_call(
        _stats_kernel,
        grid=(Bl,),
        out_shape=(jax.ShapeDtypeStruct((C, 1), jnp.float32),
                   jax.ShapeDtypeStruct((C, 1), jnp.float32)),
        in_specs=[pl.BlockSpec((1, C, HW), lambda b: (b, 0, 0))],
        out_specs=(pl.BlockSpec((C, 1), lambda b: (0, 0)),
                   pl.BlockSpec((C, 1), lambda b: (0, 0))),
        compiler_params=pltpu.CompilerParams(
            dimension_semantics=("arbitrary",)),
    )(x3)
    if axis is not None:
        ch_sum = lax.psum(ch_sum, axis)
        ch_sq = lax.psum(ch_sq, axis)

    mean = ch_sum / N
    var = jnp.maximum((ch_sq - N * mean * mean) / (N - 1), 0.0)
    scale = 1.0 / (jnp.sqrt(var) + 1e-9)
    neg_bias = -mean

    logabsdet = jnp.sum(jnp.log(jnp.sum(jnp.abs(matrix), axis=0)))
    ld_const = HW * (jnp.sum(jnp.log(jnp.abs(scale))) + logabsdet)

    pT = matrix.T.astype(jnp.bfloat16)
    w1T = w1.reshape(9 * ns, hid).T.astype(jnp.bfloat16)
    w2T = w2.T.astype(jnp.bfloat16)
    w3r = jnp.concatenate([w3[..., 0::2], w3[..., 1::2]], -1)
    w3T = jnp.transpose(w3r, (0, 3, 1, 2)).reshape(3 * 2 * co, 3 * hid)
    w3T = w3T.astype(jnp.bfloat16)
    b3r = jnp.concatenate([b3[0::2], b3[1::2]]).reshape(2 * co, 1)

    def const_spec(shape):
        return pl.BlockSpec(shape, lambda b, _s=len(shape): (0,) * _s)

    G = 4
    while Bl % G:
        G //= 2
    y3, ld_cpl = pl.pallas_call(
        functools.partial(_glow_kernel, H, W, ns, G),
        grid=(Bl // G,),
        out_shape=(jax.ShapeDtypeStruct((Bl, C, HW), jnp.float32),
                   jax.ShapeDtypeStruct((Bl, 1, 1), jnp.float32)),
        in_specs=[
            pl.BlockSpec((G, C, HW), lambda b: (b, 0, 0)),
            const_spec((C, 1)),
            const_spec((C, 1)),
            const_spec((C, C)),
            const_spec((hid, 9 * ns)),
            const_spec((hid, 1)),
            const_spec((hid, hid)),
            const_spec((hid, 1)),
            const_spec((3 * 2 * co, 3 * hid)),
            const_spec((2 * co, 1)),
        ],
        out_specs=(pl.BlockSpec((G, C, HW), lambda b: (b, 0, 0)),
                   pl.BlockSpec((G, 1, 1), lambda b: (b, 0, 0))),
        compiler_params=pltpu.CompilerParams(
            dimension_semantics=("arbitrary",),
            vmem_limit_bytes=100 * 1024 * 1024),
    )(x3, neg_bias, scale, pT, w1T, b1.reshape(hid, 1), w2T,
      b2.reshape(hid, 1), w3T, b3r)

    log_det = ld_const * jnp.ones((Bl,), jnp.float32) + ld_cpl[:, 0, 0]
    return y3, log_det


def kernel(x, matrix, w1, b1, w2, b2, w3, b3):
    B, C, H, W = x.shape
    x3 = x.reshape(B, C, H * W)

    y3, log_det = _forward(x3, matrix, w1, b1, w2, b2, w3, b3,
                           n_shards=1, axis=None)
    return y3.reshape(B, C, H, W), log_det


# speedup vs baseline: 1.8077x; 1.0590x over previous
import functools

import numpy as np
import jax
import jax.numpy as jnp
from jax import lax
from jax.experimental import pallas as pl
from jax.experimental.pallas import tpu as pltpu
from jax.sharding import Mesh, PartitionSpec as P

try:
    from jax import shard_map as _shard_map
except ImportError:
    from jax.experimental.shard_map import shard_map as _shard_map


def _rot(a, k):
    if k == 0:
        return a
    n = a.shape[1]
    k = k % n
    return jnp.concatenate([a[:, k:], a[:, :k]], axis=1)


def _stats_kernel(x_ref, sum_ref, sq_ref):
    @pl.when(pl.program_id(0) == 0)
    def _():
        sum_ref[...] = jnp.zeros_like(sum_ref)
        sq_ref[...] = jnp.zeros_like(sq_ref)

    x = x_ref[0]
    sum_ref[...] = sum_ref[...] + jnp.sum(x, axis=1, keepdims=True)
    sq_ref[...] = sq_ref[...] + jnp.sum(x * x, axis=1, keepdims=True)


def _glow_kernel(H, W, ns, G,
                 x_ref, nb_ref, sc_ref, pT_ref,
                 w1_ref, b1_ref, w2_ref, b2_ref, w3_ref, b3_ref,
                 y_ref, ld_ref):
    C = x_ref.shape[1]
    HW = x_ref.shape[2]
    co = C - ns
    n3 = 2 * co
    NW = G * HW

    pix = lax.broadcasted_iota(jnp.int32, (1, NW), 1)
    pin = pix - (pix // HW) * HW
    py = pin // W
    px = pin - py * W
    vrow = {ky: (py + ky >= 0) & (py + ky < H) for ky in (-1, 0, 1)}
    vcol = {kx: (px + kx >= 0) & (px + kx < W) for kx in (-1, 0, 1)}
    taps = [(dy - 1, dx - 1) for dy in range(3) for dx in range(3)]
    valids = [vrow[ky] & vcol[kx] for (ky, kx) in taps]
    zero_b = jnp.zeros((), jnp.bfloat16)

    xw = jnp.concatenate([x_ref[g] for g in range(G)], axis=1)
    z = (xw + nb_ref[...]) * sc_ref[...]
    zc = jnp.dot(pT_ref[...], z.astype(jnp.bfloat16),
                 preferred_element_type=jnp.float32)
    z_b = zc[ns:C, :]

    za = zc[0:ns, :].astype(jnp.bfloat16)
    cols = [jnp.where(v, _rot(za, ky * W + kx), zero_b)
            for (ky, kx), v in zip(taps, valids)]
    col = jnp.concatenate(cols, axis=0)
    h1 = jnp.dot(w1_ref[...], col, preferred_element_type=jnp.float32)
    h1 = jnp.maximum(h1 + b1_ref[...], 0.0)

    h2 = jnp.dot(w2_ref[...], h1.astype(jnp.bfloat16),
                 preferred_element_type=jnp.float32)
    h2 = jnp.maximum(h2 + b2_ref[...], 0.0)

    h2b = h2.astype(jnp.bfloat16)
    colx = jnp.concatenate(
        [jnp.where(vcol[kx], _rot(h2b, kx), zero_b) if kx else h2b
         for kx in (-1, 0, 1)], axis=0)
    part = jnp.dot(w3_ref[...], colx,
                   preferred_element_type=jnp.float32)
    acc = jnp.broadcast_to(b3_ref[...], (n3, NW))
    for iy, ky in enumerate((-1, 0, 1)):
        pj = _rot(part[iy * n3:(iy + 1) * n3, :], ky * W)
        acc = acc + jnp.where(vrow[ky], pj, 0.0)

    s = jax.nn.sigmoid(acc[0:co, :] + 2.0)
    t = acc[co:n3, :]
    yb = s * z_b + t
    ls = jnp.log(jnp.abs(s))
    for g in range(G):
        lo, hi = g * HW, (g + 1) * HW
        y_ref[g, 0:ns, :] = zc[0:ns, lo:hi]
        y_ref[g, ns:C, :] = yb[:, lo:hi]
        ld_ref[g] = jnp.sum(ls[:, lo:hi], keepdims=True)


def _forward(x3, matrix, w1, b1, w2, b2, w3, b3, n_shards, axis):
    Bl, C, HW = x3.shape
    H = W = int(round(HW ** 0.5))
    N = n_shards * Bl * HW
    ns = C // 2
    co = C - ns
    hid = w1.shape[-1]

    ch_sum, ch_sq = pl.pallas_call(
        _stats_kernel,
        grid=(Bl,),
        out_shape=(jax.ShapeDtypeStruct((C, 1), jnp.float32),
                   jax.ShapeDtypeStruct((C, 1), jnp.float32)),
        in_specs=[pl.BlockSpec((1, C, HW), lambda b: (b, 0, 0))],
        out_specs=(pl.BlockSpec((C, 1), lambda b: (0, 0)),
                   pl.BlockSpec((C, 1), lambda b: (0, 0))),
        compiler_params=pltpu.CompilerParams(
            dimension_semantics=("arbitrary",)),
    )(x3)
    if axis is not None:
        ch_sum = lax.psum(ch_sum, axis)
        ch_sq = lax.psum(ch_sq, axis)

    mean = ch_sum / N
    var = jnp.maximum((ch_sq - N * mean * mean) / (N - 1), 0.0)
    scale = 1.0 / (jnp.sqrt(var) + 1e-9)
    neg_bias = -mean

    logabsdet = jnp.sum(jnp.log(jnp.sum(jnp.abs(matrix), axis=0)))
    ld_const = HW * (jnp.sum(jnp.log(jnp.abs(scale))) + logabsdet)

    pT = matrix.T.astype(jnp.bfloat16)
    w1T = w1.reshape(9 * ns, hid).T.astype(jnp.bfloat16)
    w2T = w2.T.astype(jnp.bfloat16)
    w3r = jnp.concatenate([w3[..., 0::2], w3[..., 1::2]], -1)
    w3T = jnp.transpose(w3r, (0, 3, 1, 2)).reshape(3 * 2 * co, 3 * hid)
    w3T = w3T.astype(jnp.bfloat16)
    b3r = jnp.concatenate([b3[0::2], b3[1::2]]).reshape(2 * co, 1)

    def const_spec(shape):
        return pl.BlockSpec(shape, lambda b, _s=len(shape): (0,) * _s)

    G = 4
    while Bl % G:
        G //= 2
    y3, ld_cpl = pl.pallas_call(
        functools.partial(_glow_kernel, H, W, ns, G),
        grid=(Bl // G,),
        out_shape=(jax.ShapeDtypeStruct((Bl, C, HW), jnp.float32),
                   jax.ShapeDtypeStruct((Bl, 1, 1), jnp.float32)),
        in_specs=[
            pl.BlockSpec((G, C, HW), lambda b: (b, 0, 0)),
            const_spec((C, 1)),
            const_spec((C, 1)),
            const_spec((C, C)),
            const_spec((hid, 9 * ns)),
            const_spec((hid, 1)),
            const_spec((hid, hid)),
            const_spec((hid, 1)),
            const_spec((3 * 2 * co, 3 * hid)),
            const_spec((2 * co, 1)),
        ],
        out_specs=(pl.BlockSpec((G, C, HW), lambda b: (b, 0, 0)),
                   pl.BlockSpec((G, 1, 1), lambda b: (b, 0, 0))),
        compiler_params=pltpu.CompilerParams(
            dimension_semantics=("arbitrary",),
            vmem_limit_bytes=56 * 1024 * 1024),
    )(x3, neg_bias, scale, pT, w1T, b1.reshape(hid, 1), w2T,
      b2.reshape(hid, 1), w3T, b3r)

    log_det = ld_const * jnp.ones((Bl,), jnp.float32) + ld_cpl[:, 0, 0]
    return y3, log_det


def kernel(x, matrix, w1, b1, w2, b2, w3, b3):
    B, C, H, W = x.shape
    x3 = x.reshape(B, C, H * W)

    y3, log_det = _forward(x3, matrix, w1, b1, w2, b2, w3, b3,
                           n_shards=1, axis=None)
    return y3.reshape(B, C, H, W), log_det


# speedup vs baseline: 1.8360x; 1.0156x over previous
import functools

import jax
import jax.numpy as jnp
from jax import lax
from jax.experimental import pallas as pl
from jax.experimental.pallas import tpu as pltpu


def _rot(a, k):
    if k == 0:
        return a
    n = a.shape[1]
    k = k % n
    return jnp.concatenate([a[:, k:], a[:, :k]], axis=1)


def _dotT(w, x):
    return lax.dot_general(w, x, (((0,), (0,)), ((), ())),
                           preferred_element_type=jnp.float32)


def _stats_kernel(x_ref, sum_ref, sq_ref):
    @pl.when(pl.program_id(0) == 0)
    def _():
        sum_ref[...] = jnp.zeros_like(sum_ref)
        sq_ref[...] = jnp.zeros_like(sq_ref)

    x = x_ref[0]
    sum_ref[...] = sum_ref[...] + jnp.sum(x, axis=1, keepdims=True)
    sq_ref[...] = sq_ref[...] + jnp.sum(x * x, axis=1, keepdims=True)


def _glow_kernel(H, W, ns, G, N,
                 x_ref, sum_ref, sq_ref, m_ref,
                 w1_ref, b1_ref, w2_ref, b2_ref, w3_ref, b3_ref,
                 y_ref, ld_ref):
    C = x_ref.shape[1]
    HW = x_ref.shape[2]
    co = C - ns
    n3 = 2 * co
    NW = G * HW

    mean = sum_ref[...] * (1.0 / N)
    var = jnp.maximum((sq_ref[...] - N * mean * mean) / (N - 1), 0.0)
    sc = 1.0 / (jnp.sqrt(var) + 1e-9)
    nb = -mean

    colabs = jnp.sum(jnp.abs(m_ref[...].astype(jnp.float32)), axis=0,
                     keepdims=True)
    ld_const = HW * (jnp.sum(jnp.log(jnp.abs(sc))) +
                     jnp.sum(jnp.log(colabs)))

    pix = lax.broadcasted_iota(jnp.int32, (1, NW), 1)
    pin = pix - (pix // HW) * HW
    py = pin // W
    px = pin - py * W
    vrow = {ky: (py + ky >= 0) & (py + ky < H) for ky in (-1, 0, 1)}
    vcol = {kx: (px + kx >= 0) & (px + kx < W) for kx in (-1, 0, 1)}
    taps = [(dy - 1, dx - 1) for dy in range(3) for dx in range(3)]
    valids = [vrow[ky] & vcol[kx] for (ky, kx) in taps]
    zero_b = jnp.zeros((), jnp.bfloat16)

    xw = jnp.concatenate([x_ref[g] for g in range(G)], axis=1)
    z = (xw + nb) * sc
    zc = _dotT(m_ref[...], z.astype(jnp.bfloat16))
    z_b = zc[ns:C, :]

    za = zc[0:ns, :].astype(jnp.bfloat16)
    cols = [jnp.where(v, _rot(za, ky * W + kx), zero_b)
            for (ky, kx), v in zip(taps, valids)]
    col = jnp.concatenate(cols, axis=0)
    h1 = _dotT(w1_ref[...], col)
    h1 = jnp.maximum(h1 + b1_ref[...], 0.0)

    h2 = _dotT(w2_ref[...], h1.astype(jnp.bfloat16))
    h2 = jnp.maximum(h2 + b2_ref[...], 0.0)

    h2b = h2.astype(jnp.bfloat16)
    colx = jnp.concatenate(
        [jnp.where(vcol[kx], _rot(h2b, kx), zero_b) if kx else h2b
         for kx in (-1, 0, 1)], axis=0)
    part = jnp.dot(w3_ref[...], colx,
                   preferred_element_type=jnp.float32)
    acc = jnp.broadcast_to(b3_ref[...], (n3, NW))
    for iy, ky in enumerate((-1, 0, 1)):
        pj = _rot(part[iy * n3:(iy + 1) * n3, :], ky * W)
        acc = acc + jnp.where(vrow[ky], pj, 0.0)

    s = jax.nn.sigmoid(acc[0:co, :])
    t = acc[co:n3, :]
    yb = s * z_b + t
    ls = jnp.log(jnp.abs(s))
    for g in range(G):
        lo, hi = g * HW, (g + 1) * HW
        y_ref[g, 0:ns, :] = zc[0:ns, lo:hi]
        y_ref[g, ns:C, :] = yb[:, lo:hi]
        ld_ref[g] = ld_const + jnp.sum(ls[:, lo:hi], keepdims=True)


def kernel(x, matrix, w1, b1, w2, b2, w3, b3):
    B, C, H, W = x.shape
    HW = H * W
    N = B * HW
    ns = C // 2
    co = C - ns
    hid = w1.shape[-1]
    x3 = x.reshape(B, C, HW)

    ch_sum, ch_sq = pl.pallas_call(
        _stats_kernel,
        grid=(B,),
        out_shape=(jax.ShapeDtypeStruct((C, 1), jnp.float32),
                   jax.ShapeDtypeStruct((C, 1), jnp.float32)),
        in_specs=[pl.BlockSpec((1, C, HW), lambda b: (b, 0, 0))],
        out_specs=(pl.BlockSpec((C, 1), lambda b: (0, 0)),
                   pl.BlockSpec((C, 1), lambda b: (0, 0))),
        compiler_params=pltpu.CompilerParams(
            dimension_semantics=("arbitrary",)),
    )(x3)

    mb = matrix.astype(jnp.bfloat16)
    w1r = w1.reshape(9 * ns, hid).astype(jnp.bfloat16)
    w2b = w2.astype(jnp.bfloat16)
    w3r = jnp.concatenate([w3[..., 0::2], w3[..., 1::2]], -1)
    w3T = jnp.transpose(w3r, (0, 3, 1, 2)).reshape(3 * 2 * co, 3 * hid)
    w3T = w3T.astype(jnp.bfloat16)
    b3r = jnp.concatenate([b3[0::2] + 2.0, b3[1::2]]).reshape(2 * co, 1)

    def const_spec(shape):
        return pl.BlockSpec(shape, lambda b, _s=len(shape): (0,) * _s)

    G = 4
    while B % G:
        G //= 2
    y3, ld = pl.pallas_call(
        functools.partial(_glow_kernel, H, W, ns, G, N),
        grid=(B // G,),
        out_shape=(jax.ShapeDtypeStruct((B, C, HW), jnp.float32),
                   jax.ShapeDtypeStruct((B, 1, 1), jnp.float32)),
        in_specs=[
            pl.BlockSpec((G, C, HW), lambda b: (b, 0, 0)),
            const_spec((C, 1)),
            const_spec((C, 1)),
            const_spec((C, C)),
            const_spec((9 * ns, hid)),
            const_spec((hid, 1)),
            const_spec((hid, hid)),
            const_spec((hid, 1)),
            const_spec((3 * 2 * co, 3 * hid)),
            const_spec((2 * co, 1)),
        ],
        out_specs=(pl.BlockSpec((G, C, HW), lambda b: (b, 0, 0)),
                   pl.BlockSpec((G, 1, 1), lambda b: (b, 0, 0))),
        compiler_params=pltpu.CompilerParams(
            dimension_semantics=("arbitrary",),
            vmem_limit_bytes=56 * 1024 * 1024),
    )(x3, ch_sum, ch_sq, mb, w1r, b1.reshape(hid, 1), w2b,
      b2.reshape(hid, 1), w3T, b3r)

    return y3.reshape(B, C, H, W), ld[:, 0, 0]


# speedup vs baseline: 1.9813x; 1.0792x over previous
import functools

import jax
import jax.numpy as jnp
from jax import lax
from jax.experimental import pallas as pl
from jax.experimental.pallas import tpu as pltpu


def _rot(a, k):
    if k == 0:
        return a
    n = a.shape[1]
    k = k % n
    return jnp.concatenate([a[:, k:], a[:, :k]], axis=1)


def _dotT(w, x):
    return lax.dot_general(w, x, (((0,), (0,)), ((), ())),
                           preferred_element_type=jnp.float32)


def _stats_kernel(x_ref, sum_ref, sq_ref):
    @pl.when(pl.program_id(0) == 0)
    def _():
        sum_ref[...] = jnp.zeros_like(sum_ref)
        sq_ref[...] = jnp.zeros_like(sq_ref)

    x = x_ref[...]
    sum_ref[...] = sum_ref[...] + jnp.sum(x, axis=(0, 2))[:, None]
    sq_ref[...] = sq_ref[...] + jnp.sum(x * x, axis=(0, 2))[:, None]


def _glow_kernel(H, W, ns, G, N,
                 x_ref, sum_ref, sq_ref, m_ref,
                 w1_ref, b1_ref, w2_ref, b2_ref, w3_ref, b3_ref,
                 y_ref, ld_ref):
    C = x_ref.shape[1]
    HW = x_ref.shape[2]
    co = C - ns
    n3 = 2 * co
    NW = G * HW

    mean = sum_ref[...] * (1.0 / N)
    var = jnp.maximum((sq_ref[...] - N * mean * mean) / (N - 1), 0.0)
    sc = 1.0 / (jnp.sqrt(var) + 1e-9)
    nb = -mean

    colabs = jnp.sum(jnp.abs(m_ref[...].astype(jnp.float32)), axis=0,
                     keepdims=True)
    ld_const = HW * (jnp.sum(jnp.log(jnp.abs(sc))) +
                     jnp.sum(jnp.log(colabs)))

    pix = lax.broadcasted_iota(jnp.int32, (1, NW), 1)
    pin = pix - (pix // HW) * HW
    py = pin // W
    px = pin - py * W
    vrow = {ky: (py + ky >= 0) & (py + ky < H) for ky in (-1, 0, 1)}
    vcol = {kx: (px + kx >= 0) & (px + kx < W) for kx in (-1, 0, 1)}
    taps = [(dy - 1, dx - 1) for dy in range(3) for dx in range(3)]
    valids = [vrow[ky] & vcol[kx] for (ky, kx) in taps]
    zero_b = jnp.zeros((), jnp.bfloat16)

    xw = jnp.concatenate([x_ref[g] for g in range(G)], axis=1)
    z = (xw + nb) * sc
    zc = _dotT(m_ref[...], z.astype(jnp.bfloat16))
    z_b = zc[ns:C, :]

    za = zc[0:ns, :].astype(jnp.bfloat16)
    cols = [jnp.where(v, _rot(za, ky * W + kx), zero_b)
            for (ky, kx), v in zip(taps, valids)]
    col = jnp.concatenate(cols, axis=0)
    h1 = _dotT(w1_ref[...], col)
    h1 = jnp.maximum(h1 + b1_ref[...], 0.0)

    h2 = _dotT(w2_ref[...], h1.astype(jnp.bfloat16))
    h2 = jnp.maximum(h2 + b2_ref[...], 0.0)

    h2b = h2.astype(jnp.bfloat16)
    colx = jnp.concatenate(
        [jnp.where(vcol[kx], _rot(h2b, kx), zero_b) if kx else h2b
         for kx in (-1, 0, 1)], axis=0)
    part = jnp.dot(w3_ref[...], colx,
                   preferred_element_type=jnp.float32)
    acc = jnp.broadcast_to(b3_ref[...], (n3, NW))
    for iy, ky in enumerate((-1, 0, 1)):
        pj = _rot(part[iy * n3:(iy + 1) * n3, :], ky * W)
        acc = acc + jnp.where(vrow[ky], pj, 0.0)

    s = jax.nn.sigmoid(acc[0:co, :])
    t = acc[co:n3, :]
    yb = s * z_b + t
    ls = jnp.log(jnp.abs(s))
    for g in range(G):
        lo, hi = g * HW, (g + 1) * HW
        y_ref[g, 0:ns, :] = zc[0:ns, lo:hi]
        y_ref[g, ns:C, :] = yb[:, lo:hi]
        ld_ref[g] = ld_const + jnp.sum(ls[:, lo:hi], keepdims=True)


def kernel(x, matrix, w1, b1, w2, b2, w3, b3):
    B, C, H, W = x.shape
    HW = H * W
    N = B * HW
    ns = C // 2
    co = C - ns
    hid = w1.shape[-1]
    x3 = x.reshape(B, C, HW)

    Gs = 8
    while B % Gs:
        Gs //= 2
    ch_sum, ch_sq = pl.pallas_call(
        _stats_kernel,
        grid=(B // Gs,),
        out_shape=(jax.ShapeDtypeStruct((C, 1), jnp.float32),
                   jax.ShapeDtypeStruct((C, 1), jnp.float32)),
        in_specs=[pl.BlockSpec((Gs, C, HW), lambda b: (b, 0, 0))],
        out_specs=(pl.BlockSpec((C, 1), lambda b: (0, 0)),
                   pl.BlockSpec((C, 1), lambda b: (0, 0))),
        compiler_params=pltpu.CompilerParams(
            dimension_semantics=("arbitrary",)),
    )(x3)

    mb = matrix.astype(jnp.bfloat16)
    w1r = w1.reshape(9 * ns, hid).astype(jnp.bfloat16)
    w2b = w2.astype(jnp.bfloat16)
    w3r = jnp.concatenate([w3[..., 0::2], w3[..., 1::2]], -1)
    w3T = jnp.transpose(w3r, (0, 3, 1, 2)).reshape(3 * 2 * co, 3 * hid)
    w3T = w3T.astype(jnp.bfloat16)
    b3r = jnp.concatenate([b3[0::2] + 2.0, b3[1::2]]).reshape(2 * co, 1)

    def const_spec(shape):
        return pl.BlockSpec(shape, lambda b, _s=len(shape): (0,) * _s)

    G = 4
    while B % G:
        G //= 2
    y3, ld = pl.pallas_call(
        functools.partial(_glow_kernel, H, W, ns, G, N),
        grid=(B // G,),
        out_shape=(jax.ShapeDtypeStruct((B, C, HW), jnp.float32),
                   jax.ShapeDtypeStruct((B, 1, 1), jnp.float32)),
        in_specs=[
            pl.BlockSpec((G, C, HW), lambda b: (b, 0, 0)),
            const_spec((C, 1)),
            const_spec((C, 1)),
            const_spec((C, C)),
            const_spec((9 * ns, hid)),
            const_spec((hid, 1)),
            const_spec((hid, hid)),
            const_spec((hid, 1)),
            const_spec((3 * 2 * co, 3 * hid)),
            const_spec((2 * co, 1)),
        ],
        out_specs=(pl.BlockSpec((G, C, HW), lambda b: (b, 0, 0)),
                   pl.BlockSpec((G, 1, 1), lambda b: (b, 0, 0))),
        compiler_params=pltpu.CompilerParams(
            dimension_semantics=("arbitrary",),
            vmem_limit_bytes=56 * 1024 * 1024),
    )(x3, ch_sum, ch_sq, mb, w1r, b1.reshape(hid, 1), w2b,
      b2.reshape(hid, 1), w3T, b3r)

    return y3.reshape(B, C, H, W), ld[:, 0, 0]


# speedup vs baseline: 1.9831x; 1.0009x over previous
import functools

import jax
import jax.numpy as jnp
from jax import lax
from jax.experimental import pallas as pl
from jax.experimental.pallas import tpu as pltpu


def _rot(a, k):
    if k == 0:
        return a
    n = a.shape[1]
    k = k % n
    return jnp.concatenate([a[:, k:], a[:, :k]], axis=1)


def _dotT(w, x):
    return lax.dot_general(w, x, (((0,), (0,)), ((), ())),
                           preferred_element_type=jnp.float32)


def _stats_kernel(x_ref, sum_ref, sq_ref):
    @pl.when(pl.program_id(0) == 0)
    def _():
        sum_ref[...] = jnp.zeros_like(sum_ref)
        sq_ref[...] = jnp.zeros_like(sq_ref)

    x = x_ref[...]
    sum_ref[...] = sum_ref[...] + jnp.sum(x, axis=(0, 2))[:, None]
    sq_ref[...] = sq_ref[...] + jnp.sum(x * x, axis=(0, 2))[:, None]


def _glow_kernel(H, W, ns, G, N,
                 x_ref, sum_ref, sq_ref, m_ref,
                 w1_ref, b1_ref, w2_ref, b2_ref, w3_ref, b3_ref,
                 y_ref, ld_ref):
    C = x_ref.shape[1]
    HW = x_ref.shape[2]
    co = C - ns
    n3 = 2 * co
    NW = G * HW

    mean = sum_ref[...] * (1.0 / N)
    var = jnp.maximum((sq_ref[...] - N * mean * mean) / (N - 1), 0.0)
    sc = 1.0 / (jnp.sqrt(var) + 1e-9)
    nb = -mean

    colabs = jnp.sum(jnp.abs(m_ref[...].astype(jnp.float32)), axis=0,
                     keepdims=True)
    ld_const = HW * (jnp.sum(jnp.log(jnp.abs(sc))) +
                     jnp.sum(jnp.log(colabs)))

    pix = lax.broadcasted_iota(jnp.int32, (1, NW), 1)
    pin = pix - (pix // HW) * HW
    py = pin // W
    px = pin - py * W
    vrow = {ky: (py + ky >= 0) & (py + ky < H) for ky in (-1, 0, 1)}
    vcol = {kx: (px + kx >= 0) & (px + kx < W) for kx in (-1, 0, 1)}
    taps = [(dy - 1, dx - 1) for dy in range(3) for dx in range(3)]
    valids = [vrow[ky] & vcol[kx] for (ky, kx) in taps]
    zero_b = jnp.zeros((), jnp.bfloat16)

    zbf = jnp.concatenate(
        [((x_ref[g] + nb) * sc).astype(jnp.bfloat16) for g in range(G)],
        axis=1)
    zc = _dotT(m_ref[...], zbf)
    z_b = zc[ns:C, :]

    za = zc[0:ns, :].astype(jnp.bfloat16)
    cols = [za if (ky, kx) == (0, 0)
            else jnp.where(v, _rot(za, ky * W + kx), zero_b)
            for (ky, kx), v in zip(taps, valids)]
    col = jnp.concatenate(cols, axis=0)
    h1 = _dotT(w1_ref[...], col)
    h1 = jnp.maximum(h1 + b1_ref[...], 0.0)

    h2 = _dotT(w2_ref[...], h1.astype(jnp.bfloat16))
    h2 = jnp.maximum(h2 + b2_ref[...], 0.0)

    h2b = h2.astype(jnp.bfloat16)
    colx = jnp.concatenate(
        [jnp.where(vcol[kx], _rot(h2b, kx), zero_b) if kx else h2b
         for kx in (-1, 0, 1)], axis=0)
    part = jnp.dot(w3_ref[...], colx,
                   preferred_element_type=jnp.float32)
    acc = b3_ref[...] + part[n3:2 * n3, :]
    for iy, ky in ((0, -1), (2, 1)):
        pj = _rot(part[iy * n3:(iy + 1) * n3, :], ky * W)
        acc = acc + jnp.where(vrow[ky], pj, 0.0)

    s = jax.nn.sigmoid(acc[0:co, :])
    t = acc[co:n3, :]
    yb = s * z_b + t
    ls = jnp.log(s)
    for g in range(G):
        lo, hi = g * HW, (g + 1) * HW
        y_ref[g, 0:ns, :] = zc[0:ns, lo:hi]
        y_ref[g, ns:C, :] = yb[:, lo:hi]
        ld_ref[g] = ld_const + jnp.sum(ls[:, lo:hi], keepdims=True)


def kernel(x, matrix, w1, b1, w2, b2, w3, b3):
    B, C, H, W = x.shape
    HW = H * W
    N = B * HW
    ns = C // 2
    co = C - ns
    hid = w1.shape[-1]
    x3 = x.reshape(B, C, HW)

    Gs = 8
    while B % Gs:
        Gs //= 2
    ch_sum, ch_sq = pl.pallas_call(
        _stats_kernel,
        grid=(B // Gs,),
        out_shape=(jax.ShapeDtypeStruct((C, 1), jnp.float32),
                   jax.ShapeDtypeStruct((C, 1), jnp.float32)),
        in_specs=[pl.BlockSpec((Gs, C, HW), lambda b: (b, 0, 0))],
        out_specs=(pl.BlockSpec((C, 1), lambda b: (0, 0)),
                   pl.BlockSpec((C, 1), lambda b: (0, 0))),
        compiler_params=pltpu.CompilerParams(
            dimension_semantics=("arbitrary",)),
    )(x3)

    mb = matrix.astype(jnp.bfloat16)
    w1r = w1.reshape(9 * ns, hid).astype(jnp.bfloat16)
    w2b = w2.astype(jnp.bfloat16)
    w3r = jnp.concatenate([w3[..., 0::2], w3[..., 1::2]], -1)
    w3T = jnp.transpose(w3r, (0, 3, 1, 2)).reshape(3 * 2 * co, 3 * hid)
    w3T = w3T.astype(jnp.bfloat16)
    b3r = jnp.concatenate([b3[0::2] + 2.0, b3[1::2]]).reshape(2 * co, 1)

    def const_spec(shape):
        return pl.BlockSpec(shape, lambda b, _s=len(shape): (0,) * _s)

    G = 4
    while B % G:
        G //= 2
    y3, ld = pl.pallas_call(
        functools.partial(_glow_kernel, H, W, ns, G, N),
        grid=(B // G,),
        out_shape=(jax.ShapeDtypeStruct((B, C, HW), jnp.float32),
                   jax.ShapeDtypeStruct((B, 1, 1), jnp.float32)),
        in_specs=[
            pl.BlockSpec((G, C, HW), lambda b: (b, 0, 0)),
            const_spec((C, 1)),
            const_spec((C, 1)),
            const_spec((C, C)),
            const_spec((9 * ns, hid)),
            const_spec((hid, 1)),
            const_spec((hid, hid)),
            const_spec((hid, 1)),
            const_spec((3 * 2 * co, 3 * hid)),
            const_spec((2 * co, 1)),
        ],
        out_specs=(pl.BlockSpec((G, C, HW), lambda b: (b, 0, 0)),
                   pl.BlockSpec((G, 1, 1), lambda b: (b, 0, 0))),
        compiler_params=pltpu.CompilerParams(
            dimension_semantics=("arbitrary",),
            vmem_limit_bytes=56 * 1024 * 1024),
    )(x3, ch_sum, ch_sq, mb, w1r, b1.reshape(hid, 1), w2b,
      b2.reshape(hid, 1), w3T, b3r)

    return y3.reshape(B, C, H, W), ld[:, 0, 0]
